# baseline (speedup 1.0000x reference)
import functools

import jax
import jax.numpy as jnp
from jax import lax
from jax.experimental import pallas as pl
from jax.experimental.pallas import tpu as pltpu
from jax.experimental.pallas import tpu_sc as plsc

N = 10000
D = 128
E = 320000
SIDE_MAX = 4800
SIDE_PAD = 4864
EPS = 1e-5

NC, NS = 2, 16
NW = NC * NS
EW = E // NW
K = 125
NCHUNK = EW // K
STRIPE = SIDE_PAD // NS

B = 1000
NB = N // B
BW = 600
NBW = SIDE_MAX // BW
HPAD = 14848


def _prep_body(kb_ref, e0_ref, e1_ref, cnt_ref, g1_ref, g2_ref):
    c0 = jnp.sum((kb_ref[...] == 0).astype(jnp.int32))
    cnt_ref[0] = c0
    c1 = N - c0
    g1_ref[...] = c0 + jnp.clip(e1_ref[...], 0, c1 - 1)
    g2 = jnp.clip(e0_ref[...], 0, c0 - 1)
    g2_ref[...] = jnp.where(g2 < 0, g2 + N, g2)


_prep = pl.pallas_call(
    _prep_body,
    out_shape=(
        jax.ShapeDtypeStruct((1,), jnp.int32),
        jax.ShapeDtypeStruct((E // D, D), jnp.int32),
        jax.ShapeDtypeStruct((E // D, D), jnp.int32),
    ),
    in_specs=[pl.BlockSpec(memory_space=pltpu.VMEM)] * 3,
    out_specs=(
        pl.BlockSpec(memory_space=pltpu.SMEM),
        pl.BlockSpec(memory_space=pltpu.VMEM),
        pl.BlockSpec(memory_space=pltpu.VMEM),
    ),
)


def _sc_agg_body(table, gidx, sidx, zeros, out, gidx_a, sidx_a, rows_v, acc,
                 sem0, sem1, sem2, sem3):
    c = lax.axis_index("c")
    s = lax.axis_index("s")
    w = s * NC + c
    pltpu.sync_copy(gidx.at[w], gidx_a)
    pltpu.sync_copy(sidx.at[w], sidx_a)

    sems = (sem0, sem1, sem2, sem3)

    def start_gather(ci, b):
        pltpu.async_copy(table.at[gidx_a.at[ci]], rows_v.at[b], sems[b])

    def start_scatter(ci, b):
        pltpu.async_copy(rows_v.at[b], acc.at[sidx_a.at[ci]], sems[b],
                         add=True)

    def wait_sem(b):
        pltpu.make_async_copy(table.at[gidx_a.at[0]], rows_v.at[b],
                              sems[b]).wait()

    start_gather(0, 0)
    start_gather(1, 1)
    pltpu.sync_copy(zeros.at[pl.ds(s * STRIPE, STRIPE)],
                    acc.at[pl.ds(s * STRIPE, STRIPE)])
    plsc.subcore_barrier()

    wait_sem(0)
    start_scatter(0, 0)
    start_gather(2, 2)
    wait_sem(1)
    start_scatter(1, 1)
    start_gather(3, 3)

    def quad(i4, carry):
        c0 = 2 + i4 * 4
        for k in range(4):
            ci = c0 + k
            b = (2 + k) % 4
            bp = k
            wait_sem(b)
            start_scatter(ci, b)
            wait_sem(bp)
            start_gather(ci + 2, bp)
        return carry

    lax.fori_loop(0, (NCHUNK - 4) // 4, quad, 0)

    for ci in (NCHUNK - 2, NCHUNK - 1):
        b = ci % 4
        wait_sem(b)
        start_scatter(ci, b)
        wait_sem((b + 2) % 4)
    wait_sem((NCHUNK - 2) % 4)
    wait_sem((NCHUNK - 1) % 4)

    plsc.subcore_barrier()
    pltpu.sync_copy(acc.at[pl.ds(s * STRIPE, STRIPE)],
                    out.at[c, pl.ds(s * STRIPE, STRIPE)])


def _make_sc_agg():
    return pl.kernel(
        _sc_agg_body,
        out_type=jax.ShapeDtypeStruct((NC, SIDE_PAD, D), jnp.float32),
        mesh=plsc.VectorSubcoreMesh(core_axis_name="c", subcore_axis_name="s",
                                    num_cores=NC, num_subcores=NS),
        scratch_types=[
            pltpu.VMEM((NCHUNK, K), jnp.int32),
            pltpu.VMEM((NCHUNK, K), jnp.int32),
            pltpu.VMEM((4, K, D), jnp.float32),
            pltpu.VMEM_SHARED((SIDE_PAD, D), jnp.float32),
            pltpu.SemaphoreType.DMA,
            pltpu.SemaphoreType.DMA,
            pltpu.SemaphoreType.DMA,
            pltpu.SemaphoreType.DMA,
        ],
    )


def _dense_body(side, final, cnt_ref, xprev_ref, p_ref, *rest):
    if final:
        xs_ref = rest[0]
        rest = rest[1:]
    w1_ref, b1_ref, w2_ref, b2_ref, wcx_ref, wcz_ref, bc_ref, out_ref, h_ref \
        = rest

    c0 = cnt_ref[0]
    if side == 0:
        base = 0
        lo, hi = jnp.int32(0), c0
    else:
        base = c0
        lo, hi = c0, jnp.int32(N)
    cnt = (hi - lo).astype(jnp.float32)

    def row_mask(j):
        rid = j * B + lax.broadcasted_iota(jnp.int32, (B, 1), 0)
        return (rid >= lo) & (rid < hi)

    b1 = b1_ref[...]
    for j in range(NB):
        h_ref[pl.ds(j * B, B), :] = jnp.broadcast_to(b1, (B, D))
    w1 = w1_ref[...]
    for j in range(NBW):
        blk = p_ref[0, pl.ds(j * BW, BW), :] + p_ref[1, pl.ds(j * BW, BW), :]
        h_ref[pl.ds(base + j * BW, BW), :] = (
            jnp.dot(blk, w1, preferred_element_type=jnp.float32) + b1)

    s = jnp.zeros((1, D), jnp.float32)
    q = jnp.zeros((1, D), jnp.float32)
    for j in range(NB):
        h = h_ref[pl.ds(j * B, B), :]
        m = row_mask(j)
        s = s + jnp.sum(jnp.where(m, h, 0.0), axis=0, keepdims=True)
        q = q + jnp.sum(jnp.where(m, h * h, 0.0), axis=0, keepdims=True)
    mu1 = s / cnt
    inv1 = lax.rsqrt(q / cnt - mu1 * mu1 + EPS)

    w2 = w2_ref[...]
    b2 = b2_ref[...]
    s = jnp.zeros((1, D), jnp.float32)
    q = jnp.zeros((1, D), jnp.float32)
    for j in range(NB):
        h = h_ref[pl.ds(j * B, B), :]
        bn = jnp.maximum((h - mu1) * inv1, 0.0)
        h2 = jnp.dot(bn, w2, preferred_element_type=jnp.float32) + b2
        h_ref[pl.ds(j * B, B), :] = h2
        m = row_mask(j)
        s = s + jnp.sum(jnp.where(m, h2, 0.0), axis=0, keepdims=True)
        q = q + jnp.sum(jnp.where(m, h2 * h2, 0.0), axis=0, keepdims=True)
    mu2 = s / cnt
    inv2 = lax.rsqrt(q / cnt - mu2 * mu2 + EPS)

    wcx = wcx_ref[...]
    wcz = wcz_ref[...]
    bc = bc_ref[...]
    s = jnp.zeros((1, D), jnp.float32)
    q = jnp.zeros((1, D), jnp.float32)
    for j in range(NB):
        h = h_ref[pl.ds(j * B, B), :]
        bn = jnp.maximum((h - mu2) * inv2, 0.0)
        xp = xprev_ref[pl.ds(j * B, B), :]
        h3 = (jnp.dot(xp, wcx, preferred_element_type=jnp.float32)
              + jnp.dot(bn, wcz, preferred_element_type=jnp.float32) + bc)
        h_ref[pl.ds(j * B, B), :] = h3
        m = row_mask(j)
        s = s + jnp.sum(jnp.where(m, h3, 0.0), axis=0, keepdims=True)
        q = q + jnp.sum(jnp.where(m, h3 * h3, 0.0), axis=0, keepdims=True)
    mu3 = s / cnt
    inv3 = lax.rsqrt(q / cnt - mu3 * mu3 + EPS)

    for j in range(NB):
        h = h_ref[pl.ds(j * B, B), :]
        bn = jnp.maximum((h - mu3) * inv3, 0.0)
        xp = xprev_ref[pl.ds(j * B, B), :]
        o = jnp.where(row_mask(j), bn, xp)
        if final:
            o = xs_ref[pl.ds(j * B, B), :] + o
        out_ref[pl.ds(j * B, B), :] = o


def _make_dense(side, final):
    n_in = 3 + (1 if final else 0) + 7
    return pl.pallas_call(
        functools.partial(_dense_body, side, final),
        out_shape=jax.ShapeDtypeStruct((N, D), jnp.float32),
        in_specs=([pl.BlockSpec(memory_space=pltpu.SMEM)]
                  + [pl.BlockSpec(memory_space=pltpu.VMEM)] * (n_in - 1)),
        out_specs=pl.BlockSpec(memory_space=pltpu.VMEM),
        scratch_shapes=[pltpu.VMEM((HPAD, D), jnp.float32)],
    )


def _combiner_weights(p):
    (w1, b1), (w2, b2) = p["nn1"]
    (wc, bc), = p["combine"]
    return (w1, b1.reshape(1, D), w2, b2.reshape(1, D),
            wc[:D], wc[D:], bc.reshape(1, D))


def kernel(xs, k_batch, bipartites_list, x, params):
    lp = params["layers"][0]
    wa = _combiner_weights(lp["combine1"][0])
    wb = _combiner_weights(lp["combine2"][0])

    e0 = bipartites_list[0, 0].astype(jnp.int32)
    e1 = bipartites_list[0, 1].astype(jnp.int32)
    kb = k_batch.astype(jnp.int32).reshape(625, 16)

    cnt, g1, g2 = _prep(kb, e0.reshape(E // D, D), e1.reshape(E // D, D))
    zeros = jnp.zeros((SIDE_PAD, D), jnp.float32)
    sc_agg = _make_sc_agg()

    idx3 = (NW, NCHUNK, K)
    p_a = sc_agg(xs, g1.reshape(idx3), e0.reshape(idx3), zeros)
    out_a = _make_dense(0, False)(cnt, xs, p_a, *wa)
    p_b = sc_agg(out_a, g2.reshape(idx3), e1.reshape(idx3), zeros)
    out_b = _make_dense(1, True)(cnt, out_a, p_b, xs, *wb)
    return out_b


# speedup vs baseline: 1.0111x; 1.0111x over previous
import functools

import jax
import jax.numpy as jnp
from jax import lax
from jax.experimental import pallas as pl
from jax.experimental.pallas import tpu as pltpu
from jax.experimental.pallas import tpu_sc as plsc

N = 10000
D = 128
E = 320000
SIDE_MAX = 4800
SIDE_PAD = 4864
EPS = 1e-5

NC, NS = 2, 16
NW = NC * NS
EW = E // NW
K = 125
NCHUNK = EW // K
STRIPE = SIDE_PAD // NS

B = 1000
NB = N // B
BW = 600
NBW = SIDE_MAX // BW
HPAD = 14848


def _prep_body(kb_ref, e0_ref, e1_ref, cnt_ref, g1_ref, g2_ref):
    c0 = jnp.sum((kb_ref[...] == 0).astype(jnp.int32))
    cnt_ref[0] = c0
    c1 = N - c0
    g1_ref[...] = c0 + jnp.clip(e1_ref[...], 0, c1 - 1)
    g2 = jnp.clip(e0_ref[...], 0, c0 - 1)
    g2_ref[...] = jnp.where(g2 < 0, g2 + N, g2)


_prep = pl.pallas_call(
    _prep_body,
    out_shape=(
        jax.ShapeDtypeStruct((1,), jnp.int32),
        jax.ShapeDtypeStruct((E // D, D), jnp.int32),
        jax.ShapeDtypeStruct((E // D, D), jnp.int32),
    ),
    in_specs=[pl.BlockSpec(memory_space=pltpu.VMEM)] * 3,
    out_specs=(
        pl.BlockSpec(memory_space=pltpu.SMEM),
        pl.BlockSpec(memory_space=pltpu.VMEM),
        pl.BlockSpec(memory_space=pltpu.VMEM),
    ),
)


def _sc_agg_body(table, gidx, sidx, zeros, out, gidx_a, sidx_a, rows_v, acc,
                 sem0, sem1):
    c = lax.axis_index("c")
    s = lax.axis_index("s")
    w = s * NC + c
    pltpu.sync_copy(gidx.at[w], gidx_a)
    pltpu.sync_copy(sidx.at[w], sidx_a)

    sems = (sem0, sem1)

    def start_gather(ci, b):
        pltpu.async_copy(table.at[gidx_a.at[ci]], rows_v.at[b], sems[b])

    def wait_gather(b):
        pltpu.make_async_copy(table.at[gidx_a.at[0]], rows_v.at[b],
                              sems[b]).wait()

    def scatter(ci, b):
        pltpu.sync_copy(rows_v.at[b], acc.at[sidx_a.at[ci]], add=True)

    start_gather(0, 0)
    pltpu.sync_copy(zeros.at[pl.ds(s * STRIPE, STRIPE)],
                    acc.at[pl.ds(s * STRIPE, STRIPE)])
    plsc.subcore_barrier()


    def pair(i2, carry):
        ci = i2 * 2
        start_gather(ci + 1, 1)
        wait_gather(0)
        scatter(ci, 0)
        start_gather(ci + 2, 0)
        wait_gather(1)
        scatter(ci + 1, 1)
        return carry

    lax.fori_loop(0, NCHUNK // 2 - 1, pair, 0)
    start_gather(NCHUNK - 1, 1)
    wait_gather(0)
    scatter(NCHUNK - 2, 0)
    wait_gather(1)
    scatter(NCHUNK - 1, 1)

    plsc.subcore_barrier()
    pltpu.sync_copy(acc.at[pl.ds(s * STRIPE, STRIPE)],
                    out.at[c, pl.ds(s * STRIPE, STRIPE)])


def _make_sc_agg():
    return pl.kernel(
        _sc_agg_body,
        out_type=jax.ShapeDtypeStruct((NC, SIDE_PAD, D), jnp.float32),
        mesh=plsc.VectorSubcoreMesh(core_axis_name="c", subcore_axis_name="s",
                                    num_cores=NC, num_subcores=NS),
        scratch_types=[
            pltpu.VMEM((NCHUNK, K), jnp.int32),
            pltpu.VMEM((NCHUNK, K), jnp.int32),
            pltpu.VMEM((2, K, D), jnp.float32),
            pltpu.VMEM_SHARED((SIDE_PAD, D), jnp.float32),
            pltpu.SemaphoreType.DMA,
            pltpu.SemaphoreType.DMA,
        ],
    )


def _dense_body(side, final, cnt_ref, xprev_ref, p_ref, *rest):
    if final:
        xs_ref = rest[0]
        rest = rest[1:]
    w1_ref, b1_ref, w2_ref, b2_ref, wcx_ref, wcz_ref, bc_ref, out_ref, h_ref \
        = rest

    c0 = cnt_ref[0]
    if side == 0:
        base = 0
        lo, hi = jnp.int32(0), c0
    else:
        base = c0
        lo, hi = c0, jnp.int32(N)
    cnt = (hi - lo).astype(jnp.float32)

    def row_mask(j):
        rid = j * B + lax.broadcasted_iota(jnp.int32, (B, 1), 0)
        return (rid >= lo) & (rid < hi)

    b1 = b1_ref[...]
    for j in range(NB):
        h_ref[pl.ds(j * B, B), :] = jnp.broadcast_to(b1, (B, D))
    w1 = w1_ref[...]
    for j in range(NBW):
        blk = p_ref[0, pl.ds(j * BW, BW), :] + p_ref[1, pl.ds(j * BW, BW), :]
        h_ref[pl.ds(base + j * BW, BW), :] = (
            jnp.dot(blk, w1, preferred_element_type=jnp.float32) + b1)

    s = jnp.zeros((1, D), jnp.float32)
    q = jnp.zeros((1, D), jnp.float32)
    for j in range(NB):
        h = h_ref[pl.ds(j * B, B), :]
        m = row_mask(j)
        s = s + jnp.sum(jnp.where(m, h, 0.0), axis=0, keepdims=True)
        q = q + jnp.sum(jnp.where(m, h * h, 0.0), axis=0, keepdims=True)
    mu1 = s / cnt
    inv1 = lax.rsqrt(q / cnt - mu1 * mu1 + EPS)

    w2 = w2_ref[...]
    b2 = b2_ref[...]
    s = jnp.zeros((1, D), jnp.float32)
    q = jnp.zeros((1, D), jnp.float32)
    for j in range(NB):
        h = h_ref[pl.ds(j * B, B), :]
        bn = jnp.maximum((h - mu1) * inv1, 0.0)
        h2 = jnp.dot(bn, w2, preferred_element_type=jnp.float32) + b2
        h_ref[pl.ds(j * B, B), :] = h2
        m = row_mask(j)
        s = s + jnp.sum(jnp.where(m, h2, 0.0), axis=0, keepdims=True)
        q = q + jnp.sum(jnp.where(m, h2 * h2, 0.0), axis=0, keepdims=True)
    mu2 = s / cnt
    inv2 = lax.rsqrt(q / cnt - mu2 * mu2 + EPS)

    wcx = wcx_ref[...]
    wcz = wcz_ref[...]
    bc = bc_ref[...]
    s = jnp.zeros((1, D), jnp.float32)
    q = jnp.zeros((1, D), jnp.float32)
    for j in range(NB):
        h = h_ref[pl.ds(j * B, B), :]
        bn = jnp.maximum((h - mu2) * inv2, 0.0)
        xp = xprev_ref[pl.ds(j * B, B), :]
        h3 = (jnp.dot(xp, wcx, preferred_element_type=jnp.float32)
              + jnp.dot(bn, wcz, preferred_element_type=jnp.float32) + bc)
        h_ref[pl.ds(j * B, B), :] = h3
        m = row_mask(j)
        s = s + jnp.sum(jnp.where(m, h3, 0.0), axis=0, keepdims=True)
        q = q + jnp.sum(jnp.where(m, h3 * h3, 0.0), axis=0, keepdims=True)
    mu3 = s / cnt
    inv3 = lax.rsqrt(q / cnt - mu3 * mu3 + EPS)

    for j in range(NB):
        h = h_ref[pl.ds(j * B, B), :]
        bn = jnp.maximum((h - mu3) * inv3, 0.0)
        xp = xprev_ref[pl.ds(j * B, B), :]
        o = jnp.where(row_mask(j), bn, xp)
        if final:
            o = xs_ref[pl.ds(j * B, B), :] + o
        out_ref[pl.ds(j * B, B), :] = o


def _make_dense(side, final):
    n_in = 3 + (1 if final else 0) + 7
    return pl.pallas_call(
        functools.partial(_dense_body, side, final),
        out_shape=jax.ShapeDtypeStruct((N, D), jnp.float32),
        in_specs=([pl.BlockSpec(memory_space=pltpu.SMEM)]
                  + [pl.BlockSpec(memory_space=pltpu.VMEM)] * (n_in - 1)),
        out_specs=pl.BlockSpec(memory_space=pltpu.VMEM),
        scratch_shapes=[pltpu.VMEM((HPAD, D), jnp.float32)],
    )


def _combiner_weights(p):
    (w1, b1), (w2, b2) = p["nn1"]
    (wc, bc), = p["combine"]
    return (w1, b1.reshape(1, D), w2, b2.reshape(1, D),
            wc[:D], wc[D:], bc.reshape(1, D))


def kernel(xs, k_batch, bipartites_list, x, params):
    lp = params["layers"][0]
    wa = _combiner_weights(lp["combine1"][0])
    wb = _combiner_weights(lp["combine2"][0])

    e0 = bipartites_list[0, 0].astype(jnp.int32)
    e1 = bipartites_list[0, 1].astype(jnp.int32)
    kb = k_batch.astype(jnp.int32).reshape(625, 16)

    cnt, g1, g2 = _prep(kb, e0.reshape(E // D, D), e1.reshape(E // D, D))
    zeros = jnp.zeros((SIDE_PAD, D), jnp.float32)
    sc_agg = _make_sc_agg()

    idx3 = (NW, NCHUNK, K)
    p_a = sc_agg(xs, g1.reshape(idx3), e0.reshape(idx3), zeros)
    out_a = _make_dense(0, False)(cnt, xs, p_a, *wa)
    p_b = sc_agg(out_a, g2.reshape(idx3), e1.reshape(idx3), zeros)
    out_b = _make_dense(1, True)(cnt, out_a, p_b, xs, *wb)
    return out_b


# speedup vs baseline: 1.0159x; 1.0048x over previous
import functools

import jax
import jax.numpy as jnp
from jax import lax
from jax.experimental import pallas as pl
from jax.experimental.pallas import tpu as pltpu
from jax.experimental.pallas import tpu_sc as plsc

N = 10000
D = 128
E = 320000
SIDE_MAX = 4800
SIDE_PAD = 4864
EPS = 1e-5

NC, NS = 2, 16
NW = NC * NS
EW = E // NW
K = 125
NCHUNK = EW // K
STRIPE = SIDE_PAD // NS

B = 1000
NB = N // B
BW = 600
NBW = SIDE_MAX // BW
HPAD = 14848


def _prep_body(kb_ref, e0_ref, e1_ref, cnt_ref, g1_ref, g2_ref):
    c0 = jnp.sum((kb_ref[...] == 0).astype(jnp.int32))
    cnt_ref[0] = c0
    c1 = N - c0
    g1_ref[...] = c0 + jnp.clip(e1_ref[...], 0, c1 - 1)
    g2 = jnp.clip(e0_ref[...], 0, c0 - 1)
    g2_ref[...] = jnp.where(g2 < 0, g2 + N, g2)


_prep = pl.pallas_call(
    _prep_body,
    out_shape=(
        jax.ShapeDtypeStruct((1,), jnp.int32),
        jax.ShapeDtypeStruct((NW * NCHUNK, K), jnp.int32),
        jax.ShapeDtypeStruct((NW * NCHUNK, K), jnp.int32),
    ),
    in_specs=[pl.BlockSpec(memory_space=pltpu.VMEM)] * 3,
    out_specs=(
        pl.BlockSpec(memory_space=pltpu.SMEM),
        pl.BlockSpec(memory_space=pltpu.VMEM),
        pl.BlockSpec(memory_space=pltpu.VMEM),
    ),
)


def _sc_agg_body(table, gidx, sidx, zeros, out, gidx_a, sidx_a, rows_v, acc,
                 sem0, sem1):
    c = lax.axis_index("c")
    s = lax.axis_index("s")
    w = s * NC + c
    pltpu.sync_copy(gidx.at[w], gidx_a)
    pltpu.sync_copy(sidx.at[w], sidx_a)

    sems = (sem0, sem1)

    def start_gather(ci, b):
        pltpu.async_copy(table.at[gidx_a.at[ci]], rows_v.at[b], sems[b])

    def wait_gather(b):
        pltpu.make_async_copy(table.at[gidx_a.at[0]], rows_v.at[b],
                              sems[b]).wait()

    def scatter(ci, b):
        pltpu.sync_copy(rows_v.at[b], acc.at[sidx_a.at[ci]], add=True)

    start_gather(0, 0)
    pltpu.sync_copy(zeros.at[pl.ds(s * STRIPE, STRIPE)],
                    acc.at[pl.ds(s * STRIPE, STRIPE)])
    plsc.subcore_barrier()


    def pair(i2, carry):
        ci = i2 * 2
        start_gather(ci + 1, 1)
        wait_gather(0)
        scatter(ci, 0)
        start_gather(ci + 2, 0)
        wait_gather(1)
        scatter(ci + 1, 1)
        return carry

    lax.fori_loop(0, NCHUNK // 2 - 1, pair, 0)
    start_gather(NCHUNK - 1, 1)
    wait_gather(0)
    scatter(NCHUNK - 2, 0)
    wait_gather(1)
    scatter(NCHUNK - 1, 1)

    plsc.subcore_barrier()
    pltpu.sync_copy(acc.at[pl.ds(s * STRIPE, STRIPE)],
                    out.at[c, pl.ds(s * STRIPE, STRIPE)])


def _make_sc_agg():
    return pl.kernel(
        _sc_agg_body,
        out_type=jax.ShapeDtypeStruct((NC, SIDE_PAD, D), jnp.float32),
        mesh=plsc.VectorSubcoreMesh(core_axis_name="c", subcore_axis_name="s",
                                    num_cores=NC, num_subcores=NS),
        scratch_types=[
            pltpu.VMEM((NCHUNK, K), jnp.int32),
            pltpu.VMEM((NCHUNK, K), jnp.int32),
            pltpu.VMEM((2, K, D), jnp.float32),
            pltpu.VMEM_SHARED((SIDE_PAD, D), jnp.float32),
            pltpu.SemaphoreType.DMA,
            pltpu.SemaphoreType.DMA,
        ],
    )


def _dense_body(side, final, cnt_ref, xprev_ref, p_ref, *rest):
    if final:
        xs_ref = rest[0]
        rest = rest[1:]
    w1_ref, b1_ref, w2_ref, b2_ref, wcx_ref, wcz_ref, bc_ref, out_ref, h_ref \
        = rest

    c0 = cnt_ref[0]
    if side == 0:
        base = 0
        lo, hi = jnp.int32(0), c0
    else:
        base = c0
        lo, hi = c0, jnp.int32(N)
    cnt = (hi - lo).astype(jnp.float32)

    def row_mask(j):
        rid = j * B + lax.broadcasted_iota(jnp.int32, (B, 1), 0)
        return (rid >= lo) & (rid < hi)

    b1 = b1_ref[...]
    for j in range(NB):
        h_ref[pl.ds(j * B, B), :] = jnp.broadcast_to(b1, (B, D))
    w1 = w1_ref[...]
    for j in range(NBW):
        blk = p_ref[0, pl.ds(j * BW, BW), :] + p_ref[1, pl.ds(j * BW, BW), :]
        h_ref[pl.ds(base + j * BW, BW), :] = (
            jnp.dot(blk, w1, preferred_element_type=jnp.float32) + b1)

    s = jnp.zeros((1, D), jnp.float32)
    q = jnp.zeros((1, D), jnp.float32)
    for j in range(NB):
        h = h_ref[pl.ds(j * B, B), :]
        m = row_mask(j)
        s = s + jnp.sum(jnp.where(m, h, 0.0), axis=0, keepdims=True)
        q = q + jnp.sum(jnp.where(m, h * h, 0.0), axis=0, keepdims=True)
    mu1 = s / cnt
    inv1 = lax.rsqrt(q / cnt - mu1 * mu1 + EPS)

    w2 = w2_ref[...]
    b2 = b2_ref[...]
    s = jnp.zeros((1, D), jnp.float32)
    q = jnp.zeros((1, D), jnp.float32)
    for j in range(NB):
        h = h_ref[pl.ds(j * B, B), :]
        bn = jnp.maximum((h - mu1) * inv1, 0.0)
        h2 = jnp.dot(bn, w2, preferred_element_type=jnp.float32) + b2
        h_ref[pl.ds(j * B, B), :] = h2
        m = row_mask(j)
        s = s + jnp.sum(jnp.where(m, h2, 0.0), axis=0, keepdims=True)
        q = q + jnp.sum(jnp.where(m, h2 * h2, 0.0), axis=0, keepdims=True)
    mu2 = s / cnt
    inv2 = lax.rsqrt(q / cnt - mu2 * mu2 + EPS)

    wcx = wcx_ref[...]
    wcz = wcz_ref[...]
    bc = bc_ref[...]
    s = jnp.zeros((1, D), jnp.float32)
    q = jnp.zeros((1, D), jnp.float32)
    for j in range(NB):
        h = h_ref[pl.ds(j * B, B), :]
        bn = jnp.maximum((h - mu2) * inv2, 0.0)
        xp = xprev_ref[pl.ds(j * B, B), :]
        h3 = (jnp.dot(xp, wcx, preferred_element_type=jnp.float32)
              + jnp.dot(bn, wcz, preferred_element_type=jnp.float32) + bc)
        h_ref[pl.ds(j * B, B), :] = h3
        m = row_mask(j)
        s = s + jnp.sum(jnp.where(m, h3, 0.0), axis=0, keepdims=True)
        q = q + jnp.sum(jnp.where(m, h3 * h3, 0.0), axis=0, keepdims=True)
    mu3 = s / cnt
    inv3 = lax.rsqrt(q / cnt - mu3 * mu3 + EPS)

    for j in range(NB):
        h = h_ref[pl.ds(j * B, B), :]
        bn = jnp.maximum((h - mu3) * inv3, 0.0)
        xp = xprev_ref[pl.ds(j * B, B), :]
        o = jnp.where(row_mask(j), bn, xp)
        if final:
            o = xs_ref[pl.ds(j * B, B), :] + o
        out_ref[pl.ds(j * B, B), :] = o


def _make_dense(side, final):
    n_in = 3 + (1 if final else 0) + 7
    return pl.pallas_call(
        functools.partial(_dense_body, side, final),
        out_shape=jax.ShapeDtypeStruct((N, D), jnp.float32),
        in_specs=([pl.BlockSpec(memory_space=pltpu.SMEM)]
                  + [pl.BlockSpec(memory_space=pltpu.VMEM)] * (n_in - 1)),
        out_specs=pl.BlockSpec(memory_space=pltpu.VMEM),
        scratch_shapes=[pltpu.VMEM((HPAD, D), jnp.float32)],
    )


def _combiner_weights(p):
    (w1, b1), (w2, b2) = p["nn1"]
    (wc, bc), = p["combine"]
    return (w1, b1.reshape(1, D), w2, b2.reshape(1, D),
            wc[:D], wc[D:], bc.reshape(1, D))


def kernel(xs, k_batch, bipartites_list, x, params):
    lp = params["layers"][0]
    wa = _combiner_weights(lp["combine1"][0])
    wb = _combiner_weights(lp["combine2"][0])

    e0 = bipartites_list[0, 0].astype(jnp.int32)
    e1 = bipartites_list[0, 1].astype(jnp.int32)
    kb = k_batch.astype(jnp.int32).reshape(625, 16)

    e0r = e0.reshape(NW * NCHUNK, K)
    e1r = e1.reshape(NW * NCHUNK, K)
    cnt, g1, g2 = _prep(kb, e0r, e1r)
    zeros = jnp.zeros((SIDE_PAD, D), jnp.float32)
    sc_agg = _make_sc_agg()

    idx3 = (NW, NCHUNK, K)
    p_a = sc_agg(xs, g1.reshape(idx3), e0r.reshape(idx3), zeros)
    out_a = _make_dense(0, False)(cnt, xs, p_a, *wa)
    p_b = sc_agg(out_a, g2.reshape(idx3), e1r.reshape(idx3), zeros)
    out_b = _make_dense(1, True)(cnt, out_a, p_b, xs, *wb)
    return out_b
